# baseline (speedup 1.0000x reference)
import jax
import jax.numpy as jnp
from jax import lax
from jax.experimental import pallas as pl
from jax.experimental.pallas import tpu as pltpu
from jax.experimental.pallas import tpu_sc as plsc

BATCH = 4096
D_IN = 8192
D_FEATURES = 1024

NUM_CORES = 2
NUM_SUBCORES = 16
NUM_WORKERS = NUM_CORES * NUM_SUBCORES
LANES = 16

R = 4
J_CHUNKS = D_IN // LANES


def _sc_body(rows_per_worker, x_hbm, s_hbm, ih_hbm, out_hbm, idx_v, s_v,
             xbuf, accs0, accs1, sem_in, sem_out0, sem_out1):
    groups = rows_per_worker // R
    cid = lax.axis_index("c")
    sid = lax.axis_index("s")
    wid = sid * NUM_CORES + cid
    base = wid * rows_per_worker

    acc_sets = (accs0, accs1)
    out_sems = (sem_out0, sem_out1)

    pltpu.sync_copy(ih_hbm, idx_v)
    pltpu.sync_copy(s_hbm, s_v)

    zero16 = jnp.zeros((LANES,), jnp.float32)

    pltpu.async_copy(x_hbm.at[pl.ds(base, R)], xbuf.at[0], sem_in)

    def outer_body(i, _):
        for b in range(2):
            g = i * 2 + b
            row0 = base + g * R
            accs = acc_sets[b]
            osem = out_sems[b]

            @pl.when(g + 1 < groups)
            def _():
                pltpu.async_copy(
                    x_hbm.at[pl.ds(row0 + R, R)], xbuf.at[1 - b], sem_in)

            @pl.when(g >= 2)
            def _():
                for r in range(R):
                    pltpu.make_async_copy(
                        accs[r], out_hbm.at[row0 - 2 * R + r], osem).wait()

            @plsc.parallel_loop(0, D_FEATURES // LANES, unroll=4)
            def _(k):
                off = k * LANES
                for r in range(R):
                    accs[r][pl.ds(off, LANES)] = zero16

            pltpu.make_async_copy(
                x_hbm.at[pl.ds(row0, R)], xbuf.at[b], sem_in).wait()

            @plsc.parallel_loop(0, J_CHUNKS, unroll=8)
            def _(jc):
                jj = jc * LANES
                idx = idx_v[pl.ds(jj, LANES)]
                sv = s_v[pl.ds(jj, LANES)]
                for r in range(R):
                    v = xbuf[b, r, pl.ds(jj, LANES)]
                    plsc.addupdate_scatter(accs[r], [idx], v * sv)

            for r in range(R):
                pltpu.async_copy(accs[r], out_hbm.at[row0 + r], osem)
        return ()

    lax.fori_loop(0, groups // 2, outer_body, ())

    last = base + rows_per_worker - 2 * R
    for b in range(2):
        for r in range(R):
            pltpu.make_async_copy(
                acc_sets[b][r], out_hbm.at[last + b * R + r],
                out_sems[b]).wait()


def _count_sketch_sc(x, s_hash, i_hash):
    import functools
    b_sc = x.shape[0]
    mesh = plsc.VectorSubcoreMesh(
        core_axis_name="c", subcore_axis_name="s",
        num_cores=NUM_CORES, num_subcores=NUM_SUBCORES,
    )
    f = pl.kernel(
        functools.partial(_sc_body, b_sc // NUM_WORKERS),
        out_type=jax.ShapeDtypeStruct((b_sc, D_FEATURES), jnp.float32),
        mesh=mesh,
        scratch_types=[
            pltpu.VMEM((D_IN,), jnp.int32),
            pltpu.VMEM((D_IN,), jnp.float32),
            pltpu.VMEM((2, R, D_IN), jnp.float32),
            [pltpu.VMEM((D_FEATURES,), jnp.float32) for _ in range(R)],
            [pltpu.VMEM((D_FEATURES,), jnp.float32) for _ in range(R)],
            pltpu.SemaphoreType.DMA,
            pltpu.SemaphoreType.DMA,
            pltpu.SemaphoreType.DMA,
        ],
        compiler_params=pltpu.CompilerParams(needs_layout_passes=False),
    )
    return f(x, s_hash, i_hash)


B_SC = 0
TC_BB = 512
TC_KB = 512
TC_NK = D_IN // TC_KB


def _tc_body(ih_ref, s_ref, x_ref, out_ref, s_mat):
    k = pl.program_id(1)
    bb = pl.program_id(0)

    @pl.when(bb == 0)
    def _():
        ih = ih_ref[0, 0, :]
        sv = s_ref[0, 0, :]
        col = jax.lax.broadcasted_iota(jnp.int32, (TC_KB, D_FEATURES), 1)
        onehot = jnp.where(col == ih[:, None], sv[:, None], 0.0)
        s_mat[k] = onehot.astype(jnp.bfloat16)

    acc = jnp.dot(x_ref[...].astype(jnp.bfloat16), s_mat[k],
                  preferred_element_type=jnp.float32)

    @pl.when(k == 0)
    def _():
        out_ref[...] = acc

    @pl.when(k != 0)
    def _():
        out_ref[...] += acc


def _tc_sketch(x, s_hash, i_hash):
    nb = x.shape[0] // TC_BB
    ih3 = i_hash.reshape(TC_NK, 1, TC_KB)
    s3 = s_hash.reshape(TC_NK, 1, TC_KB)
    return pl.pallas_call(
        _tc_body,
        grid=(nb, TC_NK),
        in_specs=[
            pl.BlockSpec((1, 1, TC_KB), lambda bb, k: (k, 0, 0)),
            pl.BlockSpec((1, 1, TC_KB), lambda bb, k: (k, 0, 0)),
            pl.BlockSpec((TC_BB, TC_KB), lambda bb, k: (bb, k)),
        ],
        out_specs=pl.BlockSpec((TC_BB, D_FEATURES), lambda bb, k: (bb, 0)),
        out_shape=jax.ShapeDtypeStruct((x.shape[0], D_FEATURES),
                                       jnp.float32),
        scratch_shapes=[
            pltpu.VMEM((TC_NK, TC_KB, D_FEATURES), jnp.bfloat16),
        ],
        compiler_params=pltpu.CompilerParams(
            dimension_semantics=("arbitrary", "arbitrary")),
    )(ih3, s3, x)


@jax.jit
def _hybrid(x2, s_f32, i_i32):
    parts = []
    if B_SC > 0:
        parts.append(_count_sketch_sc(x2[:B_SC], s_f32, i_i32))
    if B_SC < BATCH:
        parts.append(_tc_sketch(x2[B_SC:], s_f32, i_i32))
    return parts[0] if len(parts) == 1 else jnp.concatenate(parts, axis=0)


def kernel(x, s_hash, i_hash):
    original_shape = (*x.shape[:-1], D_FEATURES)
    x2 = x.reshape(-1, x.shape[-1])
    out = _hybrid(x2, s_hash.astype(jnp.float32), i_hash.astype(jnp.int32))
    return out.reshape(original_shape)
